# baseline (speedup 1.0000x reference)
import functools

import jax
import jax.numpy as jnp
from jax import lax
from jax.experimental import pallas as pl
from jax.experimental.pallas import tpu as pltpu
from jax.experimental.pallas import tpu_sc as plsc

N_SEQS = 16384
N_DOCS = 512
HIDDEN = 768
N_CLASSES = 1000
CPAD = 1008
KV = CPAD // 16
L = 16
NC, NS = 2, 16
NW = NC * NS
DPW = N_DOCS // NW
WR = 32
NEG = -3.0e38


def _tc_scores_body(x_ref, w_ref, b_ref, o_ref):
    s = lax.dot_general(
        x_ref[...], w_ref[...], (((1,), (0,)), ((), ())),
        preferred_element_type=jnp.float32) + b_ref[0, 0]
    o_ref[...] = s[:, 0]


def _tc_scores(feats, W, b2):
    brs = 4096
    return pl.pallas_call(
        _tc_scores_body,
        grid=(N_SEQS // brs,),
        in_specs=[
            pl.BlockSpec((brs, HIDDEN), lambda i: (i, 0)),
            pl.BlockSpec((HIDDEN, 1), lambda i: (0, 0)),
            pl.BlockSpec((1, 1), lambda i: (0, 0)),
        ],
        out_specs=pl.BlockSpec((brs,), lambda i: (i,)),
        out_shape=jax.ShapeDtypeStruct((N_SEQS,), jnp.float32),
    )(feats, W, b2)


def _bcast_lane(v, i):
    idx = jnp.full((L,), i, jnp.int32)
    return lax.gather(
        v, idx[:, None],
        lax.GatherDimensionNumbers(offset_dims=(), collapsed_slice_dims=(0,),
                                   start_index_map=(0,)),
        slice_sizes=(1,), mode=lax.GatherScatterMode.PROMISE_IN_BOUNDS)


def _extract(v, i):
    return jnp.sum(jnp.where(lax.iota(jnp.int32, L) == i, v, 0))


def _sc_body(sco_hbm, ids_hbm, logits_hbm, mask_hbm, out_hbm,
             ids_v, sco_v, lb0, lb1, acc_v, mbuf, obuf, sem0, sem1, osem):
    cid = lax.axis_index("c")
    sid = lax.axis_index("s")
    wid = sid * NC + cid
    lane = lax.iota(jnp.int32, L)

    pltpu.sync_copy(ids_hbm, ids_v.at[pl.ds(0, N_SEQS)])
    pltpu.sync_copy(sco_hbm, sco_v.at[pl.ds(0, N_SEQS)])
    pltpu.sync_copy(mask_hbm, mbuf.at[pl.ds(0, N_CLASSES)])

    def zk(k, c):
        co = k * 16
        mo = jnp.where(lane < N_CLASSES - co, (mbuf[pl.ds(co, 16)] - 1.0) * 1e10,
                       0.0)
        mbuf[pl.ds(co, 16)] = mo
        acc_v[pl.ds(co, 16)] = jnp.zeros((L,), jnp.float32)
        return c
    lax.fori_loop(0, KV, zk, 0)

    d0 = wid * DPW
    t0 = d0 + lane

    def lower_bound(t):
        def step(_, lh):
            lo, hi = lh
            mid = lax.shift_right_logical(lo + hi, 1)
            v = plsc.load_gather(ids_v, [mid])
            less = v < t
            return (jnp.where(less, mid + 1, lo), jnp.where(less, hi, mid))
        lo, _ = lax.fori_loop(
            0, 14, step,
            (jnp.zeros((L,), jnp.int32), jnp.full((L,), N_SEQS, jnp.int32)))
        return lo

    starts = lower_bound(t0)
    ends = lower_bound(t0 + 1)

    def wdoc(l, dvec):
        start = _extract(starts, l)
        end = _extract(ends, l)
        n = end - start
        nch = lax.shift_right_logical(n + 15, 4)

        def amax(j, mv):
            off = start + j * 16
            v = sco_v[pl.ds(off, 16)]
            return jnp.maximum(mv, jnp.where(lane < end - off, v, NEG))
        mv = lax.fori_loop(0, nch, amax, jnp.full((L,), NEG, jnp.float32))
        m = jnp.max(mv)

        def bex(j, dv):
            off = start + j * 16
            msk = lane < end - off
            v = sco_v[pl.ds(off, 16)]
            e = jnp.where(msk, jnp.exp(v - m), 0.0)
            plsc.store_scatter(sco_v, [off + lane], e, mask=msk)
            return dv + e
        dv = lax.fori_loop(0, nch, bex, jnp.zeros((L,), jnp.float32))
        return jnp.where(lane == l, jnp.sum(dv), dvec)

    denoms = lax.fori_loop(0, DPW, wdoc, jnp.zeros((L,), jnp.float32))
    scales = 1.0 / jnp.where(denoms < 0.5, 1.0, denoms)

    s_w = _extract(starts, 0)
    e_w = _extract(ends, DPW - 1)
    base = pl.multiple_of(s_w & ~7, 8)
    nwin = lax.shift_right_logical(e_w - base + WR - 1, 5)

    def rdof(c):
        return pl.multiple_of(jnp.minimum(base + c * WR, N_SEQS - WR) & ~7, 8)

    def issue(c, buf, sem):
        pltpu.async_copy(logits_hbm.at[pl.ds(rdof(c), WR), :], buf, sem)

    def drain(c, buf, sem):
        pltpu.make_async_copy(
            logits_hbm.at[pl.ds(rdof(c), WR), :], buf, sem).wait()

    def accum_range(a, b, rb, rd, buf):
        for g in range(WR // 16):
            rg = a + g * 16

            @pl.when(rg < b)
            def _(rg=rg):
                hi = jnp.minimum(b, rg + 16)
                wv = sco_v[pl.ds(rg, 16)]
                wv = jnp.where(lane < hi - rg, wv, 0.0)
                wbc = [_bcast_lane(wv, i) for i in range(16)]
                bi0 = rg - rd
                rows = [jnp.minimum(bi0 + i, WR - 1) for i in range(16)]

                def kbody(k, _k):
                    co = k * 16
                    a = acc_v[pl.ds(co, 16)]
                    for h in range(2):
                        t = [wbc[h * 8 + i] * buf[rows[h * 8 + i], pl.ds(co, 16)]
                             for i in range(8)]
                        while len(t) > 1:
                            t = [t[j] + t[j + 1] for j in range(0, len(t), 2)]
                        a = a + t[0]
                    acc_v[pl.ds(co, 16)] = a
                    return _k
                lax.fori_loop(0, KV, kbody, 0)

    def finalize(cu):
        scv = _bcast_lane(scales, cu)
        ob = pl.multiple_of(cu * CPAD, 8)

        def fk(k, c):
            co = k * 16
            obuf[pl.ds(ob + co, 16)] = acc_v[pl.ds(co, 16)] * scv + mbuf[pl.ds(co, 16)]
            acc_v[pl.ds(co, 16)] = jnp.zeros((L,), jnp.float32)
            return c
        lax.fori_loop(0, KV, fk, 0)
        pltpu.async_copy(
            obuf.at[pl.ds(ob, N_CLASSES)],
            out_hbm.at[pl.ds(pl.multiple_of((d0 + cu) * N_CLASSES, 8),
                             N_CLASSES)],
            osem)

    def process(c, buf, cu):
        rb = base + c * WR
        rd = rdof(c)
        inwin = (ends >= rb) & (ends < rb + WR)
        cnt = jnp.sum(jnp.where(inwin, 1, 0))

        def fdoc(j, cc):
            s = _extract(starts, cc)
            e = _extract(ends, cc)
            accum_range(jnp.maximum(s, rb), e, rb, rd, buf)
            finalize(cc)
            return cc + 1
        cu = lax.fori_loop(0, cnt, fdoc, cu)

        @pl.when(cu < DPW)
        def _open():
            s = _extract(starts, cu)
            accum_range(jnp.maximum(s, rb), rb + WR, rb, rd, buf)
        return cu

    @pl.when(nwin > 0)
    def _prime():
        issue(0, lb0, sem0)

    npair = lax.shift_right_logical(nwin + 1, 1)

    def pair(t, cu):
        c0 = 2 * t
        c1 = c0 + 1
        drain(c0, lb0, sem0)

        @pl.when(c1 < nwin)
        def _i1():
            issue(c1, lb1, sem1)
        cu = process(c0, lb0, cu)

        @pl.when(c0 + 2 < nwin)
        def _i2():
            issue(c0 + 2, lb0, sem0)

        def do_c1(cc):
            drain(c1, lb1, sem1)
            return process(c1, lb1, cc)
        cu = lax.cond(c1 < nwin, do_c1, lambda cc: cc, cu)
        return cu

    cur = lax.fori_loop(0, npair, pair, 0)

    def post(l, c):
        finalize(l)
        return c
    lax.fori_loop(cur, DPW, post, 0)

    def od(l, c):
        pltpu.make_async_copy(
            obuf.at[pl.ds(pl.multiple_of(l * CPAD, 8), N_CLASSES)],
            out_hbm.at[pl.ds(pl.multiple_of((d0 + l) * N_CLASSES, 8),
                             N_CLASSES)],
            osem).wait()
        return c
    lax.fori_loop(0, DPW, od, 0)


def _sc_call(scores, ids32, logits, mask):
    mesh = plsc.VectorSubcoreMesh(core_axis_name="c", subcore_axis_name="s")
    f = pl.kernel(
        _sc_body,
        mesh=mesh,
        compiler_params=pltpu.CompilerParams(needs_layout_passes=False),
        out_type=jax.ShapeDtypeStruct((N_DOCS * N_CLASSES,), jnp.float32),
        scratch_types=[
            pltpu.VMEM((N_SEQS,), jnp.int32),
            pltpu.VMEM((N_SEQS + 32,), jnp.float32),
            pltpu.VMEM((WR, N_CLASSES), jnp.float32),
            pltpu.VMEM((WR, N_CLASSES), jnp.float32),
            pltpu.VMEM((CPAD,), jnp.float32),
            pltpu.VMEM((CPAD,), jnp.float32),
            pltpu.VMEM((DPW * CPAD,), jnp.float32),
            pltpu.SemaphoreType.DMA,
            pltpu.SemaphoreType.DMA,
            pltpu.SemaphoreType.DMA,
        ],
    )
    return f(scores, ids32, logits, mask)


def kernel(seq_feats, seq_logits, segment_ids, W_attn, b_attn, doc_label_mask):
    ids32 = segment_ids.astype(jnp.int32)
    b2 = b_attn.reshape(1, 1)
    scores = _tc_scores(seq_feats, W_attn, b2)
    out1d = _sc_call(scores, ids32, seq_logits, doc_label_mask)
    return out1d.reshape(N_DOCS, N_CLASSES)


# speedup vs baseline: 1.0018x; 1.0018x over previous
import jax
import jax.numpy as jnp
from jax import lax
from jax.experimental import pallas as pl
from jax.experimental.pallas import tpu as pltpu
from jax.experimental.pallas import tpu_sc as plsc

N_SEQS = 16384
N_DOCS = 512
HIDDEN = 768
N_CLASSES = 1000
CPAD = 1008
KV = CPAD // 16
L = 16
NC, NS = 2, 16
NW = NC * NS
DPW = N_DOCS // NW
WR = 32
NEG = -3.0e38


def _tc_scores_body(x_ref, w_ref, b_ref, o_ref):
    s = lax.dot_general(
        x_ref[...], w_ref[...], (((1,), (0,)), ((), ())),
        preferred_element_type=jnp.float32) + b_ref[0, 0]
    o_ref[...] = s[:, 0]


def _tc_scores(feats, W, b2):
    brs = 4096
    return pl.pallas_call(
        _tc_scores_body,
        grid=(N_SEQS // brs,),
        in_specs=[
            pl.BlockSpec((brs, HIDDEN), lambda i: (i, 0)),
            pl.BlockSpec((HIDDEN, 1), lambda i: (0, 0)),
            pl.BlockSpec((1, 1), lambda i: (0, 0)),
        ],
        out_specs=pl.BlockSpec((brs,), lambda i: (i,)),
        out_shape=jax.ShapeDtypeStruct((N_SEQS,), jnp.float32),
    )(feats, W, b2)


def _bcast_lane(v, i):
    idx = jnp.full((L,), i, jnp.int32)
    return lax.gather(
        v, idx[:, None],
        lax.GatherDimensionNumbers(offset_dims=(), collapsed_slice_dims=(0,),
                                   start_index_map=(0,)),
        slice_sizes=(1,), mode=lax.GatherScatterMode.PROMISE_IN_BOUNDS)


def _extract(v, i):
    return jnp.sum(jnp.where(lax.iota(jnp.int32, L) == i, v, 0))


def _sc_body(sco_hbm, ids_hbm, logits_hbm, mask_hbm, out_hbm,
             ids_v, sco_v, lb0, lb1, acc_v, mbuf, obuf, sem0, sem1, osem):
    cid = lax.axis_index("c")
    sid = lax.axis_index("s")
    wid = sid * NC + cid
    lane = lax.iota(jnp.int32, L)

    pltpu.sync_copy(ids_hbm, ids_v.at[pl.ds(0, N_SEQS)])
    pltpu.sync_copy(sco_hbm, sco_v.at[pl.ds(0, N_SEQS)])
    pltpu.sync_copy(mask_hbm, mbuf.at[pl.ds(0, N_CLASSES)])

    def zk(k, c):
        co = k * 16
        mo = jnp.where(lane < N_CLASSES - co, (mbuf[pl.ds(co, 16)] - 1.0) * 1e10,
                       0.0)
        mbuf[pl.ds(co, 16)] = mo
        acc_v[pl.ds(co, 16)] = jnp.zeros((L,), jnp.float32)
        return c
    lax.fori_loop(0, KV, zk, 0)

    d0 = wid * DPW
    t0 = d0 + lane

    def lower_bound(t):
        def step(_, lh):
            lo, hi = lh
            mid = lax.shift_right_logical(lo + hi, 1)
            v = plsc.load_gather(ids_v, [mid])
            less = v < t
            return (jnp.where(less, mid + 1, lo), jnp.where(less, hi, mid))
        lo, _ = lax.fori_loop(
            0, 14, step,
            (jnp.zeros((L,), jnp.int32), jnp.full((L,), N_SEQS, jnp.int32)))
        return lo

    starts = lower_bound(t0)
    ends = lower_bound(t0 + 1)

    def wdoc(l, dvec):
        start = _extract(starts, l)
        end = _extract(ends, l)
        n = end - start
        nch = lax.shift_right_logical(n + 15, 4)

        def amax(j, mv):
            off = start + j * 16
            v = sco_v[pl.ds(off, 16)]
            return jnp.maximum(mv, jnp.where(lane < end - off, v, NEG))
        mv = lax.fori_loop(0, nch, amax, jnp.full((L,), NEG, jnp.float32))
        m = jnp.max(mv)

        def bex(j, dv):
            off = start + j * 16
            msk = lane < end - off
            v = sco_v[pl.ds(off, 16)]
            e = jnp.where(msk, jnp.exp(v - m), 0.0)
            plsc.store_scatter(sco_v, [off + lane], e, mask=msk)
            return dv + e
        dv = lax.fori_loop(0, nch, bex, jnp.zeros((L,), jnp.float32))
        return jnp.where(lane == l, jnp.sum(dv), dvec)

    denoms = lax.fori_loop(0, DPW, wdoc, jnp.zeros((L,), jnp.float32))
    scales = 1.0 / jnp.where(denoms < 0.5, 1.0, denoms)

    s_w = _extract(starts, 0)
    e_w = _extract(ends, DPW - 1)
    base = pl.multiple_of(s_w & ~7, 8)
    nwin = lax.shift_right_logical(e_w - base + WR - 1, 5)

    def rdof(c):
        return pl.multiple_of(jnp.minimum(base + c * WR, N_SEQS - WR) & ~7, 8)

    def issue(c, buf, sem):
        pltpu.async_copy(logits_hbm.at[pl.ds(rdof(c), WR), :], buf, sem)

    def drain(c, buf, sem):
        pltpu.make_async_copy(
            logits_hbm.at[pl.ds(rdof(c), WR), :], buf, sem).wait()

    def accum_range(a, b, rb, rd, buf):
        for g in range(WR // 16):
            rg = a + g * 16

            @pl.when(rg < b)
            def _(rg=rg):
                hi = jnp.minimum(b, rg + 16)
                wv = sco_v[pl.ds(rg, 16)]
                wv = jnp.where(lane < hi - rg, wv, 0.0)
                wbc = [_bcast_lane(wv, i) for i in range(16)]
                bi0 = rg - rd
                rows = [jnp.minimum(bi0 + i, WR - 1) for i in range(16)]

                def kbody(k, _k):
                    co = k * 16
                    a = acc_v[pl.ds(co, 16)]
                    for h in range(2):
                        t = [wbc[h * 8 + i] * buf[rows[h * 8 + i], pl.ds(co, 16)]
                             for i in range(8)]
                        while len(t) > 1:
                            t = [t[j] + t[j + 1] for j in range(0, len(t), 2)]
                        a = a + t[0]
                    acc_v[pl.ds(co, 16)] = a
                    return _k
                lax.fori_loop(0, KV, kbody, 0)

    def finalize(cu):
        scv = _bcast_lane(scales, cu)
        ob = pl.multiple_of(cu * CPAD, 8)

        def fk(k, c):
            co = k * 16
            obuf[pl.ds(ob + co, 16)] = acc_v[pl.ds(co, 16)] * scv + mbuf[pl.ds(co, 16)]
            acc_v[pl.ds(co, 16)] = jnp.zeros((L,), jnp.float32)
            return c
        lax.fori_loop(0, KV, fk, 0)
        pltpu.async_copy(
            obuf.at[pl.ds(ob, N_CLASSES)],
            out_hbm.at[pl.ds(pl.multiple_of((d0 + cu) * N_CLASSES, 8),
                             N_CLASSES)],
            osem)

    def process(c, buf, cu):
        rb = base + c * WR
        rd = rdof(c)
        inwin = (ends >= rb) & (ends < rb + WR)
        cnt = jnp.sum(jnp.where(inwin, 1, 0))

        def fdoc(j, cc):
            s = _extract(starts, cc)
            e = _extract(ends, cc)
            accum_range(jnp.maximum(s, rb), e, rb, rd, buf)
            finalize(cc)
            return cc + 1
        cu = lax.fori_loop(0, cnt, fdoc, cu)

        @pl.when(cu < DPW)
        def _open():
            s = _extract(starts, cu)
            accum_range(jnp.maximum(s, rb), rb + WR, rb, rd, buf)
        return cu

    @pl.when(nwin > 0)
    def _prime():
        issue(0, lb0, sem0)

    npair = lax.shift_right_logical(nwin + 1, 1)

    def pair(t, cu):
        c0 = 2 * t
        c1 = c0 + 1
        drain(c0, lb0, sem0)

        @pl.when(c1 < nwin)
        def _i1():
            issue(c1, lb1, sem1)
        cu = process(c0, lb0, cu)

        @pl.when(c0 + 2 < nwin)
        def _i2():
            issue(c0 + 2, lb0, sem0)

        def do_c1(cc):
            drain(c1, lb1, sem1)
            return process(c1, lb1, cc)
        cu = lax.cond(c1 < nwin, do_c1, lambda cc: cc, cu)
        return cu

    cur = lax.fori_loop(0, npair, pair, 0)

    def post(l, c):
        finalize(l)
        return c
    lax.fori_loop(cur, DPW, post, 0)

    def od(l, c):
        pltpu.make_async_copy(
            obuf.at[pl.ds(pl.multiple_of(l * CPAD, 8), N_CLASSES)],
            out_hbm.at[pl.ds(pl.multiple_of((d0 + l) * N_CLASSES, 8),
                             N_CLASSES)],
            osem).wait()
        return c
    lax.fori_loop(0, DPW, od, 0)


def _sc_call(scores, ids32, logits, mask):
    mesh = plsc.VectorSubcoreMesh(core_axis_name="c", subcore_axis_name="s")
    f = pl.kernel(
        _sc_body,
        mesh=mesh,
        compiler_params=pltpu.CompilerParams(needs_layout_passes=False),
        out_type=jax.ShapeDtypeStruct((N_DOCS * N_CLASSES,), jnp.float32),
        scratch_types=[
            pltpu.VMEM((N_SEQS,), jnp.int32),
            pltpu.VMEM((N_SEQS + 32,), jnp.float32),
            pltpu.VMEM((WR, N_CLASSES), jnp.float32),
            pltpu.VMEM((WR, N_CLASSES), jnp.float32),
            pltpu.VMEM((CPAD,), jnp.float32),
            pltpu.VMEM((CPAD,), jnp.float32),
            pltpu.VMEM((DPW * CPAD,), jnp.float32),
            pltpu.SemaphoreType.DMA,
            pltpu.SemaphoreType.DMA,
            pltpu.SemaphoreType.DMA,
        ],
    )
    return f(scores, ids32, logits, mask)


def kernel(seq_feats, seq_logits, segment_ids, W_attn, b_attn, doc_label_mask):
    ids32 = segment_ids.astype(jnp.int32)
    b2 = b_attn.reshape(1, 1)
    scores = _tc_scores(seq_feats, W_attn, b2)
    out1d = _sc_call(scores, ids32, seq_logits, doc_label_mask)
    return out1d.reshape(N_DOCS, N_CLASSES)
